# Initial kernel scaffold; baseline (speedup 1.0000x reference)
import jax
import jax.numpy as jnp
from jax.experimental import pallas as pl


def kernel(z, num_frames, eps, target_means, target_stds):
    raise NotImplementedError("write your pallas kernel here")


# speedup vs baseline: 2.5495x; 2.5495x over previous
import functools

import jax
import jax.numpy as jnp
from jax import lax
from jax.experimental import pallas as pl
from jax.experimental.pallas import tpu as pltpu
from jax.experimental.pallas import tpu_sc as plsc

B = 4096
T = 200
D = 32
N = B * T

NC, NS = 2, 16
NW = NC * NS
PER_W = N // NW
CHUNK = 512
NCHUNK = PER_W // CHUNK
GSUB = 128
NGS = CHUNK // GSUB


def _sc_body(z_hbm, emm_hbm, mu_hbm, sd_hbm, out_hbm,
             idx_v, emm_v, mu_v, sd_v, out_v, sem):
    wid = lax.axis_index("s") * NC + lax.axis_index("c")

    def chunk_body(c, _):
        base = pl.multiple_of(wid * PER_W + c * CHUNK, CHUNK)
        row0 = pl.multiple_of(wid * (PER_W // GSUB) + c * NGS, NGS)
        pltpu.sync_copy(z_hbm.at[pl.ds(row0, NGS)], idx_v)
        pltpu.sync_copy(emm_hbm.at[pl.ds(2 * base, 2 * CHUNK)],
                        emm_v.at[pl.ds(0, 2 * CHUNK)])
        for j in range(NGS):
            pltpu.async_copy(mu_hbm.at[idx_v.at[j]],
                             mu_v.at[pl.ds(j * GSUB, GSUB)], sem).wait()
            pltpu.async_copy(sd_hbm.at[idx_v.at[j]],
                             sd_v.at[pl.ds(j * GSUB, GSUB)], sem).wait()

        def frame(f, _):
            s = emm_v[pl.ds(2 * f, 16)]
            em_s = s[0]
            mm_s = s[1]
            for h in range(D // 16):
                sl = pl.ds(h * 16, 16)
                v = mu_v[f, sl] * mm_s + em_s * sd_v[f, sl]
                v = jnp.minimum(jnp.maximum(v, -1.0), 1.0)
                out_v[f, sl] = v
            return 0

        lax.fori_loop(0, CHUNK, frame, 0)
        pltpu.sync_copy(out_v, out_hbm.at[pl.ds(base, CHUNK)])
        return 0

    lax.fori_loop(0, NCHUNK, chunk_body, 0)


@jax.jit
def kernel(z, num_frames, eps, target_means, target_stds):
    zi = z.astype(jnp.int32)
    frame_idx = lax.broadcasted_iota(jnp.int32, (B, T), 1)
    mask = (zi != 0) & (frame_idx < num_frames.astype(jnp.int32)[:, None])
    mm = mask.astype(jnp.float32)
    em = eps * mm
    emm = jnp.stack([em.reshape(N), mm.reshape(N)], axis=-1).reshape(2 * N)

    mesh = plsc.VectorSubcoreMesh(core_axis_name="c", subcore_axis_name="s")
    run = functools.partial(
        pl.kernel,
        mesh=mesh,
        out_type=jax.ShapeDtypeStruct((N, D), jnp.float32),
        scratch_types=[
            pltpu.VMEM((NGS, GSUB), jnp.int32),
            pltpu.VMEM((2 * CHUNK + 16,), jnp.float32),
            pltpu.VMEM((CHUNK, D), jnp.float32),
            pltpu.VMEM((CHUNK, D), jnp.float32),
            pltpu.VMEM((CHUNK, D), jnp.float32),
            pltpu.SemaphoreType.DMA,
        ],
        compiler_params=pltpu.CompilerParams(use_tc_tiling_on_sc=False),
    )(_sc_body)
    out = run(zi.reshape(N // GSUB, GSUB), emm, target_means, target_stds)
    return out.reshape(B, T, D)


# speedup vs baseline: 3.5653x; 1.3985x over previous
import functools

import jax
import jax.numpy as jnp
from jax import lax
from jax.experimental import pallas as pl
from jax.experimental.pallas import tpu as pltpu
from jax.experimental.pallas import tpu_sc as plsc

B = 4096
T = 200
D = 32

NC, NS = 2, 16
NW = NC * NS
BB = 512
NSB = B // BB
NBLK = T * NSB
PER_W = NBLK // NW
GSUB = 128
NGS = BB // GSUB
NG = BB // 16


def _sc_body(z_hbm, emc_hbm, mu_hbm, sd_hbm, out_hbm,
             idx_v, emc_v, mu_v, sd_v, out_v, sem):
    wid = lax.axis_index("s") * NC + lax.axis_index("c")
    lanes = lax.iota(jnp.int32, 16)

    def block_body(k, _):
        blk = wid * PER_W + k
        t = blk // NSB
        sb = blk - t * NSB
        pltpu.sync_copy(z_hbm.at[blk], idx_v)
        pltpu.sync_copy(emc_hbm.at[blk], emc_v)
        copies = []
        for j in range(NGS):
            copies.append(pltpu.async_copy(
                mu_hbm.at[idx_v.at[j]], mu_v.at[pl.ds(j * GSUB, GSUB)], sem))
            copies.append(pltpu.async_copy(
                sd_hbm.at[idx_v.at[j]], sd_v.at[pl.ds(j * GSUB, GSUB)], sem))
        for c in copies:
            c.wait()

        def group(g, _):
            em16 = emc_v[0, pl.ds(g * 16, 16)]
            mm16 = emc_v[1, pl.ds(g * 16, 16)]
            rows = g * 16 + lanes
            bt_l = g // 8
            off = (g - bt_l * 8) * 16
            for d in range(D):
                cols = jnp.full((16,), d, jnp.int32)
                mu = plsc.load_gather(mu_v, [rows, cols])
                sd = plsc.load_gather(sd_v, [rows, cols])
                v = mu * mm16 + em16 * sd
                v = jnp.minimum(jnp.maximum(v, -1.0), 1.0)
                dt, dm = d // 8, d % 8
                out_v[dt, bt_l, pl.ds(dm * 128 + off, 16)] = v
            return 0

        lax.fori_loop(0, NG, group, 0)
        for dt in range(D // 8):
            pltpu.sync_copy(out_v.at[dt],
                            out_hbm.at[t, dt, pl.ds(sb * NGS, NGS)])
        return 0

    lax.fori_loop(0, PER_W, block_body, 0)


@jax.jit
def kernel(z, num_frames, eps, target_means, target_stds):
    zi = z.astype(jnp.int32)
    frame_idx = lax.broadcasted_iota(jnp.int32, (B, T), 1)
    mask = (zi != 0) & (frame_idx < num_frames.astype(jnp.int32)[:, None])
    mm_t = mask.T.astype(jnp.float32)
    em_t = eps.T * mm_t
    emc = jnp.stack([em_t.reshape(T, NSB, BB),
                     mm_t.reshape(T, NSB, BB)], axis=2).reshape(NBLK, 2, BB)
    z_t = zi.T.reshape(NBLK, NGS, GSUB)

    mesh = plsc.VectorSubcoreMesh(core_axis_name="c", subcore_axis_name="s")
    run = functools.partial(
        pl.kernel,
        mesh=mesh,
        out_type=jax.ShapeDtypeStruct((T, D // 8, B // GSUB, 8 * GSUB),
                                      jnp.float32),
        scratch_types=[
            pltpu.VMEM((NGS, GSUB), jnp.int32),
            pltpu.VMEM((2, BB), jnp.float32),
            pltpu.VMEM((BB, D), jnp.float32),
            pltpu.VMEM((BB, D), jnp.float32),
            pltpu.VMEM((D // 8, NGS, 8 * GSUB), jnp.float32),
            pltpu.SemaphoreType.DMA,
        ],
        compiler_params=pltpu.CompilerParams(use_tc_tiling_on_sc=False,
                                             needs_layout_passes=False),
    )(_sc_body)
    out_lin = run(z_t, emc, target_means, target_stds)
    out = (out_lin.reshape(T, D // 8, B // GSUB, 8, GSUB)
           .transpose(2, 4, 0, 1, 3).reshape(B, T, D))
    return out


# speedup vs baseline: 7.7836x; 2.1832x over previous
import functools

import jax
import jax.numpy as jnp
from jax import lax
from jax.experimental import pallas as pl
from jax.experimental.pallas import tpu as pltpu
from jax.experimental.pallas import tpu_sc as plsc

B = 4096
T = 200
D = 32

NC, NS = 2, 16
NW = NC * NS
BB = 512
NSB = B // BB
NBLK = T * NSB
PER_W = NBLK // NW
GSUB = 128
NGS = BB // GSUB
NG = BB // 16


def _sc_body(z_hbm, emc_hbm, mu_hbm, sd_hbm, out_hbm,
             idx_v, emc_v, mu_v, sd_v, out_v, sem):
    wid = lax.axis_index("s") * NC + lax.axis_index("c")
    lanes = lax.iota(jnp.int32, 16)

    def block_body(k, _):
        blk = wid * PER_W + k
        t = blk // NSB
        sb = blk - t * NSB
        pltpu.sync_copy(z_hbm.at[blk], idx_v)
        pltpu.sync_copy(emc_hbm.at[blk], emc_v)
        copies = []
        for j in range(NGS):
            copies.append(pltpu.async_copy(
                mu_hbm.at[idx_v.at[j]], mu_v.at[pl.ds(j * GSUB, GSUB)], sem))
            copies.append(pltpu.async_copy(
                sd_hbm.at[idx_v.at[j]], sd_v.at[pl.ds(j * GSUB, GSUB)], sem))
        for c in copies:
            c.wait()

        def group(g, _):
            em16 = emc_v[0, pl.ds(g * 16, 16)]
            mm16 = emc_v[1, pl.ds(g * 16, 16)]
            rows = g * 16 + lanes
            goff = (g // 8) * 1024 + (g % 8) * 16
            for d0 in (0, 16):
                for k in range(16):
                    m = (lanes + k) & 15
                    cols = m + d0
                    mu = plsc.load_gather(mu_v, [rows, cols])
                    sd = plsc.load_gather(sd_v, [rows, cols])
                    v = mu * mm16 + em16 * sd
                    v = jnp.minimum(jnp.maximum(v, -1.0), 1.0)
                    dtv = (d0 // 8) + (m >> 3)
                    inner = ((m & 7) << 7) + lanes + goff
                    plsc.store_scatter(out_v, [dtv, inner], v)
            return 0

        lax.fori_loop(0, NG, group, 0)
        for dt in range(D // 8):
            pltpu.sync_copy(out_v.at[dt],
                            out_hbm.at[t, dt, pl.ds(sb * NGS * 8 * GSUB,
                                                    NGS * 8 * GSUB)])
        return 0

    lax.fori_loop(0, PER_W, block_body, 0)


@jax.jit
def kernel(z, num_frames, eps, target_means, target_stds):
    zi = z.astype(jnp.int32)
    frame_idx = lax.broadcasted_iota(jnp.int32, (B, T), 1)
    mask = (zi != 0) & (frame_idx < num_frames.astype(jnp.int32)[:, None])
    mm_t = mask.T.astype(jnp.float32)
    em_t = eps.T * mm_t
    emc = jnp.stack([em_t.reshape(T, NSB, BB),
                     mm_t.reshape(T, NSB, BB)], axis=2).reshape(NBLK, 2, BB)
    z_t = zi.T.reshape(NBLK, NGS, GSUB)

    mesh = plsc.VectorSubcoreMesh(core_axis_name="c", subcore_axis_name="s")
    run = functools.partial(
        pl.kernel,
        mesh=mesh,
        out_type=jax.ShapeDtypeStruct((T, D // 8, B * 8), jnp.float32),
        scratch_types=[
            pltpu.VMEM((NGS, GSUB), jnp.int32),
            pltpu.VMEM((2, BB), jnp.float32),
            pltpu.VMEM((BB, D), jnp.float32),
            pltpu.VMEM((BB, D), jnp.float32),
            pltpu.VMEM((D // 8, NGS * 8 * GSUB), jnp.float32),
            pltpu.SemaphoreType.DMA,
        ],
        compiler_params=pltpu.CompilerParams(use_tc_tiling_on_sc=False,
                                             needs_layout_passes=False),
    )(_sc_body)
    out_lin = run(z_t, emc, target_means, target_stds)
    out = (out_lin.reshape(T, D // 8, B // GSUB, 8, GSUB)
           .transpose(2, 4, 0, 1, 3).reshape(B, T, D))
    return out
